# baseline (speedup 1.0000x reference)
import jax
import jax.numpy as jnp
from jax.experimental import pallas as pl


def _identity(x_ref, o_ref):
    o_ref[...] = x_ref[...]


def kernel(x, edge_index_0, edge_weight_0, edge_index_1, edge_weight_1, edge_index_2, edge_weight_2):
    h = []
    for ei, w in ((edge_index_0, edge_weight_0), (edge_index_1, edge_weight_1), (edge_index_2, edge_weight_2)):
        dst = ei[0]
        src = ei[1]
        msgs = w[:, None] * jnp.take(x, src, axis=0)
        h.append(jax.ops.segment_sum(msgs, dst, num_segments=x.shape[0]))
    h.append(x)
    out = jnp.concatenate(h, axis=1)
    return pl.pallas_call(
        _identity,
        out_shape=jax.ShapeDtypeStruct(out.shape, out.dtype),
    )(out)


# speedup vs baseline: 2.6447x; 2.6447x over previous
import functools

import jax
import jax.numpy as jnp
from jax import lax
from jax.experimental import pallas as pl
from jax.experimental.pallas import tpu as pltpu
from jax.experimental.pallas import tpu_sc as plsc

N = 10000
D = 128
E = 320000
DH = 64
NC = 2
NS = 16
UNIT = 128
UNITS_PER_CHUNK = 8
CHUNK = UNIT * UNITS_PER_CHUNK
CHUNKS_PER_TILE = 20
EPAD = NS * CHUNKS_PER_TILE * CHUNK
EROWS = EPAD // UNIT
SLAB = 640
ZROWS = 64

_mesh = plsc.VectorSubcoreMesh(core_axis_name="c", subcore_axis_name="s")

_f32 = jnp.float32
_i32 = jnp.int32


@functools.partial(
    pl.kernel,
    out_type=[jax.ShapeDtypeStruct((NC, N, DH), _f32) for _ in range(3)],
    mesh=_mesh,
    compiler_params=pltpu.CompilerParams(
        needs_layout_passes=False, use_tc_tiling_on_sc=False),
    scratch_types=[
        pltpu.VMEM((UNITS_PER_CHUNK, UNIT), _i32),
        pltpu.VMEM((UNITS_PER_CHUNK, UNIT), _i32),
        pltpu.VMEM((CHUNK,), _f32),
        pltpu.VMEM((UNIT, DH), _f32),
        pltpu.VMEM((ZROWS, DH), _f32),
        pltpu.VMEM_SHARED((N, DH), _f32),
        pltpu.VMEM_SHARED((N, DH), _f32),
    ],
)
def _disconv_sc(xcat, s0, d0, w0, s1, d1, w1, s2, d2, w2,
                o0, o1, o2,
                src_v, dst_v, w_v, rows_v, zbuf, acc0, acc1):
    c = lax.axis_index("c")
    s = lax.axis_index("s")
    cN = c * N
    row0 = jnp.minimum(s * SLAB, N - SLAB)
    zero16 = jnp.zeros((16,), _f32)

    @plsc.parallel_loop(0, ZROWS, 1, unroll=2)
    def _zfill(r):
        for q in range(DH // 16):
            zbuf[r, pl.ds(q * 16, 16)] = zero16

    def zero_acc(acc):
        for k in range(SLAB // ZROWS):
            pltpu.sync_copy(zbuf, acc.at[pl.ds(row0 + k * ZROWS, ZROWS)])

    def spmm(acc, src_h, dst_h, wgt_h):
        def chunk_body(chunk, _):
            base = (s * CHUNKS_PER_TILE + chunk) * UNITS_PER_CHUNK
            pltpu.sync_copy(src_h.at[pl.ds(base, UNITS_PER_CHUNK)], src_v)
            pltpu.sync_copy(dst_h.at[pl.ds(base, UNITS_PER_CHUNK)], dst_v)
            pltpu.sync_copy(wgt_h.at[pl.ds(base * UNIT, CHUNK)], w_v)

            @plsc.parallel_loop(0, UNITS_PER_CHUNK * UNIT, 16, unroll=2)
            def _off(i):
                r = i // UNIT
                q = i % UNIT
                src_v[r, pl.ds(q, 16)] = src_v[r, pl.ds(q, 16)] + cN

            for j in range(UNITS_PER_CHUNK):
                pltpu.sync_copy(xcat.at[src_v.at[j]], rows_v)

                @plsc.parallel_loop(0, UNIT, 1, unroll=2)
                def _scale(e, j=j):
                    wv = plsc.load_gather(
                        w_v, [jnp.full((16,), j * UNIT, _i32) + e])
                    for q in range(DH // 16):
                        rows_v[e, pl.ds(q * 16, 16)] = (
                            rows_v[e, pl.ds(q * 16, 16)] * wv)

                pltpu.sync_copy(rows_v, acc.at[dst_v.at[j]], add=True)
            return 0

        lax.fori_loop(0, CHUNKS_PER_TILE, chunk_body, 0)

    def flush(acc, out):
        pltpu.sync_copy(acc.at[pl.ds(row0, SLAB)], out.at[c, pl.ds(row0, SLAB)])

    zero_acc(acc0)
    zero_acc(acc1)
    plsc.subcore_barrier()
    spmm(acc0, s0, d0, w0)
    spmm(acc1, s1, d1, w1)
    plsc.subcore_barrier()
    flush(acc0, o0)
    flush(acc1, o1)

    zero_acc(acc0)
    plsc.subcore_barrier()
    spmm(acc0, s2, d2, w2)
    plsc.subcore_barrier()
    flush(acc0, o2)


def _pad_edges(ei, w):
    pad = EPAD - E
    src = jnp.concatenate([ei[1], jnp.zeros((pad,), _i32)]).reshape(EROWS, UNIT)
    dst = jnp.concatenate([ei[0], jnp.zeros((pad,), _i32)]).reshape(EROWS, UNIT)
    wp = jnp.concatenate([w, jnp.zeros((pad,), _f32)])
    return src, dst, wp


def kernel(x, edge_index_0, edge_weight_0, edge_index_1, edge_weight_1,
           edge_index_2, edge_weight_2):
    xcat = jnp.concatenate([x[:, :DH], x[:, DH:]], axis=0)
    s0, d0, w0 = _pad_edges(edge_index_0, edge_weight_0)
    s1, d1, w1 = _pad_edges(edge_index_1, edge_weight_1)
    s2, d2, w2 = _pad_edges(edge_index_2, edge_weight_2)
    o0, o1, o2 = _disconv_sc(xcat, s0, d0, w0, s1, d1, w1, s2, d2, w2)
    h = [jnp.concatenate([o[0], o[1]], axis=1) for o in (o0, o1, o2)]
    h.append(x)
    return jnp.concatenate(h, axis=1)


# speedup vs baseline: 3.3340x; 1.2606x over previous
import functools

import jax
import jax.numpy as jnp
from jax import lax
from jax.experimental import pallas as pl
from jax.experimental.pallas import tpu as pltpu
from jax.experimental.pallas import tpu_sc as plsc

N = 10000
D = 128
E = 320000
DH = 64
NC = 2
NS = 16
UNIT = 128
UNITS_PER_CHUNK = 8
CHUNK = UNIT * UNITS_PER_CHUNK
CHUNKS_PER_TILE = 20
EPAD = NS * CHUNKS_PER_TILE * CHUNK
EROWS = EPAD // UNIT
SLAB = 640
ZROWS = 64

_mesh = plsc.VectorSubcoreMesh(core_axis_name="c", subcore_axis_name="s")

_f32 = jnp.float32
_i32 = jnp.int32


@functools.partial(
    pl.kernel,
    out_type=[jax.ShapeDtypeStruct((NC, N, DH), _f32) for _ in range(3)],
    mesh=_mesh,
    compiler_params=pltpu.CompilerParams(
        needs_layout_passes=False, use_tc_tiling_on_sc=False),
    scratch_types=[
        pltpu.VMEM((UNITS_PER_CHUNK, UNIT), _i32),
        pltpu.VMEM((UNITS_PER_CHUNK, UNIT), _i32),
        pltpu.VMEM((CHUNK,), _f32),
        pltpu.VMEM((2, UNIT, DH), _f32),
        pltpu.VMEM((ZROWS, DH), _f32),
        pltpu.VMEM_SHARED((N, DH), _f32),
        pltpu.VMEM_SHARED((N, DH), _f32),
        pltpu.SemaphoreType.DMA,
        pltpu.SemaphoreType.DMA,
        pltpu.SemaphoreType.DMA,
        pltpu.SemaphoreType.DMA,
    ],
)
def _disconv_sc(xcat, s0, d0, w0, s1, d1, w1, s2, d2, w2,
                o0, o1, o2,
                src_v, dst_v, w_v, rows_v, zbuf, acc0, acc1,
                sem_g0, sem_g1, sem_s0, sem_s1):
    sem_g = (sem_g0, sem_g1)
    sem_s = (sem_s0, sem_s1)
    c = lax.axis_index("c")
    s = lax.axis_index("s")
    cN = c * N
    row0 = jnp.minimum(s * SLAB, N - SLAB)
    zero16 = jnp.zeros((16,), _f32)

    @plsc.parallel_loop(0, ZROWS, 1, unroll=2)
    def _zfill(r):
        for q in range(DH // 16):
            zbuf[r, pl.ds(q * 16, 16)] = zero16

    def zero_acc(acc):
        for k in range(SLAB // ZROWS):
            pltpu.sync_copy(zbuf, acc.at[pl.ds(row0 + k * ZROWS, ZROWS)])

    def spmm(acc, src_h, dst_h, wgt_h):
        def chunk_body(chunk, _):
            base = (s * CHUNKS_PER_TILE + chunk) * UNITS_PER_CHUNK
            pltpu.sync_copy(src_h.at[pl.ds(base, UNITS_PER_CHUNK)], src_v)
            pltpu.sync_copy(dst_h.at[pl.ds(base, UNITS_PER_CHUNK)], dst_v)
            pltpu.sync_copy(wgt_h.at[pl.ds(base * UNIT, CHUNK)], w_v)

            @plsc.parallel_loop(0, UNITS_PER_CHUNK * UNIT, 16, unroll=2)
            def _off(i):
                r = i // UNIT
                q = i % UNIT
                src_v[r, pl.ds(q, 16)] = src_v[r, pl.ds(q, 16)] + cN

            gd = [None, None]
            sd = [None, None]
            gd[0] = pltpu.async_copy(
                xcat.at[src_v.at[0]], rows_v.at[0], sem_g[0])
            for j in range(UNITS_PER_CHUNK):
                p = j % 2
                if j + 1 < UNITS_PER_CHUNK:
                    if sd[1 - p] is not None:
                        sd[1 - p].wait()
                        sd[1 - p] = None
                    gd[1 - p] = pltpu.async_copy(
                        xcat.at[src_v.at[j + 1]], rows_v.at[1 - p],
                        sem_g[1 - p])
                gd[p].wait()

                @plsc.parallel_loop(0, UNIT, 1, unroll=2)
                def _scale(e, j=j, p=p):
                    wv = plsc.load_gather(
                        w_v, [jnp.full((16,), j * UNIT, _i32) + e])
                    for q in range(DH // 16):
                        rows_v[p, e, pl.ds(q * 16, 16)] = (
                            rows_v[p, e, pl.ds(q * 16, 16)] * wv)

                sd[p] = pltpu.async_copy(
                    rows_v.at[p], acc.at[dst_v.at[j]], sem_s[p], add=True)
            sd[0].wait()
            sd[1].wait()
            return 0

        lax.fori_loop(0, CHUNKS_PER_TILE, chunk_body, 0)

    def flush(acc, out):
        pltpu.sync_copy(acc.at[pl.ds(row0, SLAB)], out.at[c, pl.ds(row0, SLAB)])

    zero_acc(acc0)
    zero_acc(acc1)
    plsc.subcore_barrier()
    spmm(acc0, s0, d0, w0)
    spmm(acc1, s1, d1, w1)
    plsc.subcore_barrier()
    flush(acc0, o0)
    flush(acc1, o1)

    zero_acc(acc0)
    plsc.subcore_barrier()
    spmm(acc0, s2, d2, w2)
    plsc.subcore_barrier()
    flush(acc0, o2)


def _pad_edges(ei, w):
    pad = EPAD - E
    src = jnp.concatenate([ei[1], jnp.zeros((pad,), _i32)]).reshape(EROWS, UNIT)
    dst = jnp.concatenate([ei[0], jnp.zeros((pad,), _i32)]).reshape(EROWS, UNIT)
    wp = jnp.concatenate([w, jnp.zeros((pad,), _f32)])
    return src, dst, wp


def kernel(x, edge_index_0, edge_weight_0, edge_index_1, edge_weight_1,
           edge_index_2, edge_weight_2):
    xcat = jnp.concatenate([x[:, :DH], x[:, DH:]], axis=0)
    s0, d0, w0 = _pad_edges(edge_index_0, edge_weight_0)
    s1, d1, w1 = _pad_edges(edge_index_1, edge_weight_1)
    s2, d2, w2 = _pad_edges(edge_index_2, edge_weight_2)
    o0, o1, o2 = _disconv_sc(xcat, s0, d0, w0, s1, d1, w1, s2, d2, w2)
    h = [jnp.concatenate([o[0], o[1]], axis=1) for o in (o0, o1, o2)]
    h.append(x)
    return jnp.concatenate(h, axis=1)


# speedup vs baseline: 5.2361x; 1.5705x over previous
import functools

import jax
import jax.numpy as jnp
from jax import lax
from jax.experimental import pallas as pl
from jax.experimental.pallas import tpu as pltpu
from jax.experimental.pallas import tpu_sc as plsc

N = 10000
D = 128
E = 320000
DH = 64
NC = 2
NS = 16
UNIT = 128
UNITS_PER_CHUNK = 8
CHUNK = UNIT * UNITS_PER_CHUNK
CHUNKS_PER_TILE = 20
EPAD = NS * CHUNKS_PER_TILE * CHUNK
EROWS = EPAD // UNIT
SLAB = 640
ZROWS = 64

_mesh = plsc.VectorSubcoreMesh(core_axis_name="c", subcore_axis_name="s")

_f32 = jnp.float32
_i32 = jnp.int32


@functools.partial(
    pl.kernel,
    out_type=[jax.ShapeDtypeStruct((NC, N, DH), _f32) for _ in range(3)],
    mesh=_mesh,
    compiler_params=pltpu.CompilerParams(
        needs_layout_passes=False, use_tc_tiling_on_sc=False),
    scratch_types=[
        pltpu.VMEM((UNITS_PER_CHUNK, UNIT), _i32),
        pltpu.VMEM((UNITS_PER_CHUNK, UNIT), _i32),
        pltpu.VMEM((CHUNK,), _f32),
        pltpu.VMEM((2, UNIT, DH), _f32),
        pltpu.VMEM((ZROWS, DH), _f32),
        pltpu.VMEM_SHARED((N, DH), _f32),
        pltpu.VMEM_SHARED((N, DH), _f32),
        pltpu.SemaphoreType.DMA,
        pltpu.SemaphoreType.DMA,
        pltpu.SemaphoreType.DMA,
        pltpu.SemaphoreType.DMA,
    ],
)
def _disconv_sc(xcat, s0, d0, w0, s1, d1, w1, s2, d2, w2,
                o0, o1, o2,
                src_v, dst_v, w_v, rows_v, zbuf, xtab, acc,
                sem_g0, sem_g1, sem_s0, sem_s1):
    sem_g = (sem_g0, sem_g1)
    sem_s = (sem_s0, sem_s1)
    c = lax.axis_index("c")
    s = lax.axis_index("s")
    row0 = jnp.minimum(s * SLAB, N - SLAB)
    zero16 = jnp.zeros((16,), _f32)

    def zero_acc():
        @plsc.parallel_loop(0, ZROWS, 1, unroll=2)
        def _zf(r):
            for q in range(DH // 16):
                zbuf[r, pl.ds(q * 16, 16)] = zero16

        for k in range(SLAB // ZROWS):
            pltpu.sync_copy(zbuf, acc.at[pl.ds(row0 + k * ZROWS, ZROWS)])

    def spmm(src_h, dst_h, wgt_h):
        def chunk_body(chunk, _):
            base = (s * CHUNKS_PER_TILE + chunk) * UNITS_PER_CHUNK
            pltpu.sync_copy(src_h.at[pl.ds(base, UNITS_PER_CHUNK)], src_v)
            pltpu.sync_copy(dst_h.at[pl.ds(base, UNITS_PER_CHUNK)], dst_v)
            pltpu.sync_copy(wgt_h.at[pl.ds(base * UNIT, CHUNK)], w_v)

            gd = [None, None]
            sd = [None, None]
            gd[0] = pltpu.async_copy(
                xtab.at[src_v.at[0]], rows_v.at[0], sem_g[0])
            for j in range(UNITS_PER_CHUNK):
                p = j % 2
                if j + 1 < UNITS_PER_CHUNK:
                    if sd[1 - p] is not None:
                        sd[1 - p].wait()
                        sd[1 - p] = None
                    gd[1 - p] = pltpu.async_copy(
                        xtab.at[src_v.at[j + 1]], rows_v.at[1 - p],
                        sem_g[1 - p])
                gd[p].wait()

                @plsc.parallel_loop(0, UNIT, 1, unroll=2)
                def _scale(e, j=j, p=p):
                    wv = plsc.load_gather(
                        w_v, [jnp.full((16,), j * UNIT, _i32) + e])
                    for q in range(DH // 16):
                        rows_v[p, e, pl.ds(q * 16, 16)] = (
                            rows_v[p, e, pl.ds(q * 16, 16)] * wv)

                sd[p] = pltpu.async_copy(
                    rows_v.at[p], acc.at[dst_v.at[j]], sem_s[p], add=True)
            sd[0].wait()
            sd[1].wait()
            return 0

        lax.fori_loop(0, CHUNKS_PER_TILE, chunk_body, 0)

    def flush(out):
        for k in range(SLAB // ZROWS):
            pltpu.sync_copy(acc.at[pl.ds(row0 + k * ZROWS, ZROWS)], zbuf)
        plsc.subcore_barrier()
        pltpu.sync_copy(acc.at[pl.ds(row0, SLAB)], out.at[c, pl.ds(row0, SLAB)])

    for k in range(SLAB // ZROWS):
        pltpu.sync_copy(xcat.at[pl.ds(c * N + row0 + k * ZROWS, ZROWS)], zbuf)
        pltpu.sync_copy(zbuf, xtab.at[pl.ds(row0 + k * ZROWS, ZROWS)])
    for k in range(SLAB // ZROWS):
        pltpu.sync_copy(xtab.at[pl.ds(row0 + k * ZROWS, ZROWS)], zbuf)
    zero_acc()
    plsc.subcore_barrier()

    for idx, (src_h, dst_h, wgt_h, out) in enumerate((
            (s0, d0, w0, o0), (s1, d1, w1, o1), (s2, d2, w2, o2))):
        spmm(src_h, dst_h, wgt_h)
        plsc.subcore_barrier()
        flush(out)
        if idx < 2:
            zero_acc()
            plsc.subcore_barrier()


def _pad_edges(ei, w):
    pad = EPAD - E
    src = jnp.concatenate([ei[1], jnp.zeros((pad,), _i32)]).reshape(EROWS, UNIT)
    dst = jnp.concatenate([ei[0], jnp.zeros((pad,), _i32)]).reshape(EROWS, UNIT)
    wp = jnp.concatenate([w, jnp.zeros((pad,), _f32)])
    return src, dst, wp


def kernel(x, edge_index_0, edge_weight_0, edge_index_1, edge_weight_1,
           edge_index_2, edge_weight_2):
    xcat = jnp.concatenate([x[:, :DH], x[:, DH:]], axis=0)
    s0, d0, w0 = _pad_edges(edge_index_0, edge_weight_0)
    s1, d1, w1 = _pad_edges(edge_index_1, edge_weight_1)
    s2, d2, w2 = _pad_edges(edge_index_2, edge_weight_2)
    o0, o1, o2 = _disconv_sc(xcat, s0, d0, w0, s1, d1, w1, s2, d2, w2)
    h = [jnp.concatenate([o[0], o[1]], axis=1) for o in (o0, o1, o2)]
    h.append(x)
    return jnp.concatenate(h, axis=1)
